# baseline (speedup 1.0000x reference)
import functools

import jax
import jax.numpy as jnp
from jax import lax
from jax.experimental import pallas as pl
from jax.experimental.pallas import tpu as pltpu
from jax.experimental.pallas import tpu_sc as plsc

_D = 128
_LANES = 16
_ND = _D // _LANES
_NC = 2
_NS = 16
_NW = _NC * _NS
_G = 2
_NBUF = 8
_UNROLL = 5


@functools.lru_cache(maxsize=None)
def _build(B, H, V):
    b_per_w = B // _NW
    ngroups = b_per_w // _G
    nchunks = ngroups // _NBUF
    rows_per_chunk = _NBUF * _G
    k = _G * H
    assert H % _UNROLL == 0 and ngroups % _NBUF == 0 and nchunks % 2 == 0

    mesh = plsc.VectorSubcoreMesh(core_axis_name="c", subcore_axis_name="s")

    @functools.partial(
        pl.kernel,
        out_type=jax.ShapeDtypeStruct((B, _D), jnp.float32),
        mesh=mesh,
        scratch_types=[
            pltpu.VMEM((ngroups, k), jnp.int32),
            pltpu.VMEM((_NBUF, k, _D), jnp.float32),
            pltpu.VMEM((2, rows_per_chunk, _D), jnp.float32),
            pltpu.VMEM((_D,), jnp.float32),
            pltpu.SemaphoreType.DMA,
            pltpu.SemaphoreType.DMA,
            pltpu.SemaphoreType.DMA,
            pltpu.SemaphoreType.DMA,
            pltpu.SemaphoreType.DMA,
            pltpu.SemaphoreType.DMA,
            pltpu.SemaphoreType.DMA,
            pltpu.SemaphoreType.DMA,
            pltpu.SemaphoreType.DMA,
            pltpu.SemaphoreType.DMA,
            pltpu.SemaphoreType.DMA,
        ],
    )
    def emb_sum(idx_hbm, table_hbm, bias_hbm, out_hbm,
                idx_v, buf_v, out_v, bias_v,
                sem0, sem1, sem2, sem3, sem4, sem5, sem6, sem7,
                semo0, semo1, sem_io):
        wid = lax.axis_index("s") * _NC + lax.axis_index("c")
        pltpu.async_copy(bias_hbm, bias_v, sem_io).wait()
        pltpu.async_copy(idx_hbm.at[wid], idx_v, sem_io).wait()

        sems = (sem0, sem1, sem2, sem3, sem4, sem5, sem6, sem7)
        osems = (semo0, semo1)
        for slot in range(_NBUF - 1):
            pltpu.async_copy(
                table_hbm.at[idx_v.at[slot]], buf_v.at[slot], sems[slot])

        bias_regs = tuple(
            bias_v[pl.ds(d * _LANES, _LANES)] for d in range(_ND))

        def do_chunk(j, oslot, bias_regs, first):
            if not first:
                pltpu.make_async_copy(
                    out_v.at[oslot],
                    out_hbm.at[pl.ds(wid * b_per_w, rows_per_chunk)],
                    osems[oslot]).wait()

            for b in range(_NBUF):
                g = _NBUF * j + b
                pltpu.make_async_copy(
                    table_hbm.at[idx_v.at[g]], buf_v.at[b], sems[b]).wait()

                nslot = (b + _NBUF - 1) % _NBUF
                cond = g + _NBUF - 1 < ngroups

                def _prefetch(_g=g, _ns=nslot):
                    pltpu.async_copy(
                        table_hbm.at[idx_v.at[_g + _NBUF - 1]],
                        buf_v.at[_ns], sems[_ns])

                if isinstance(cond, bool):
                    if cond:
                        _prefetch()
                else:
                    pl.when(cond)(_prefetch)

                accs = bias_regs + bias_regs

                def body(l, accs, _b=b):
                    for u in range(_UNROLL):
                        accs = tuple(
                            accs[r * _ND + d]
                            + buf_v[_b, r * H + l * _UNROLL + u,
                                    pl.ds(d * _LANES, _LANES)]
                            for r in range(_G) for d in range(_ND))
                    return accs

                accs = lax.fori_loop(0, H // _UNROLL, body, accs)
                for r in range(_G):
                    for d in range(_ND):
                        out_v[oslot, b * _G + r,
                              pl.ds(d * _LANES, _LANES)] = accs[r * _ND + d]

            pltpu.async_copy(
                out_v.at[oslot],
                out_hbm.at[pl.ds(wid * b_per_w + j * rows_per_chunk,
                                 rows_per_chunk)],
                osems[oslot])

        def chunk_pair(t, bias_regs):
            do_chunk(2 * t, 0, bias_regs, False)
            do_chunk(2 * t + 1, 1, bias_regs, False)
            return bias_regs

        do_chunk(0, 0, bias_regs, True)
        do_chunk(1, 1, bias_regs, True)
        lax.fori_loop(1, nchunks // 2, chunk_pair, bias_regs)

        for oslot in range(2):
            pltpu.make_async_copy(
                out_v.at[oslot],
                out_hbm.at[pl.ds(wid * b_per_w, rows_per_chunk)],
                osems[oslot]).wait()

    return emb_sum


def kernel(sent_a, table, bias):
    B, H = sent_a.shape
    V, D = table.shape
    assert D == _D and B % (_NW * _G) == 0
    idx = sent_a.astype(jnp.int32).reshape(_NW, (B // _NW) // _G, _G * H)
    return _build(B, H, V)(idx, table, bias)


# speedup vs baseline: 1.1635x; 1.1635x over previous
import functools

import jax
import jax.numpy as jnp
from jax import lax
from jax.experimental import pallas as pl
from jax.experimental.pallas import tpu as pltpu
from jax.experimental.pallas import tpu_sc as plsc

_D = 128
_LANES = 16
_ND = _D // _LANES
_NC = 2
_NS = 16
_NW = _NC * _NS
_NBUF = 8
_UNROLL = 5


@functools.lru_cache(maxsize=None)
def _build(B, H, V):
    b_per_w = B // _NW
    assert H % _UNROLL == 0 and b_per_w % _NBUF == 0

    mesh = plsc.VectorSubcoreMesh(core_axis_name="c", subcore_axis_name="s")

    @functools.partial(
        pl.kernel,
        out_type=jax.ShapeDtypeStruct((B, _D), jnp.float32),
        mesh=mesh,
        scratch_types=[
            pltpu.VMEM((b_per_w, H), jnp.int32),
            pltpu.VMEM((_NBUF, H, _D), jnp.float32),
            pltpu.VMEM((b_per_w, _D), jnp.float32),
            pltpu.VMEM((_D,), jnp.float32),
            pltpu.SemaphoreType.DMA,
            pltpu.SemaphoreType.DMA,
            pltpu.SemaphoreType.DMA,
            pltpu.SemaphoreType.DMA,
            pltpu.SemaphoreType.DMA,
            pltpu.SemaphoreType.DMA,
            pltpu.SemaphoreType.DMA,
            pltpu.SemaphoreType.DMA,
            pltpu.SemaphoreType.DMA,
        ],
    )
    def emb_sum(idx_hbm, table_hbm, bias_hbm, out_hbm,
                idx_v, buf_v, out_v, bias_v,
                sem0, sem1, sem2, sem3, sem4, sem5, sem6, sem7, sem_io):
        wid = lax.axis_index("s") * _NC + lax.axis_index("c")
        pltpu.async_copy(bias_hbm, bias_v, sem_io).wait()
        pltpu.async_copy(
            idx_hbm.at[pl.ds(wid * b_per_w, b_per_w)], idx_v, sem_io).wait()

        sems = (sem0, sem1, sem2, sem3, sem4, sem5, sem6, sem7)
        for slot in range(_NBUF - 1):
            pltpu.async_copy(
                table_hbm.at[idx_v.at[slot]], buf_v.at[slot], sems[slot])

        bias_regs = tuple(
            bias_v[pl.ds(d * _LANES, _LANES)] for d in range(_ND))

        def ring(j, bias_regs):
            for b in range(_NBUF):
                g = _NBUF * j + b
                pltpu.make_async_copy(
                    table_hbm.at[idx_v.at[g]], buf_v.at[b], sems[b]).wait()

                nslot = (b + _NBUF - 1) % _NBUF

                @pl.when(g + _NBUF - 1 < b_per_w)
                def _prefetch(_g=g, _ns=nslot):
                    pltpu.async_copy(
                        table_hbm.at[idx_v.at[_g + _NBUF - 1]],
                        buf_v.at[_ns], sems[_ns])

                accs = bias_regs

                def body(l, accs, _b=b):
                    for u in range(_UNROLL):
                        accs = tuple(
                            accs[d] + buf_v[_b, l * _UNROLL + u,
                                            pl.ds(d * _LANES, _LANES)]
                            for d in range(_ND))
                    return accs

                accs = lax.fori_loop(0, H // _UNROLL, body, accs)
                for d in range(_ND):
                    out_v[g, pl.ds(d * _LANES, _LANES)] = accs[d]
            return bias_regs

        lax.fori_loop(0, b_per_w // _NBUF, ring, bias_regs)
        pltpu.async_copy(
            out_v, out_hbm.at[pl.ds(wid * b_per_w, b_per_w)], sem_io).wait()

    return emb_sum


def kernel(sent_a, table, bias):
    B, H = sent_a.shape
    V, D = table.shape
    assert D == _D and B % _NW == 0
    return _build(B, H, V)(sent_a.astype(jnp.int32), table, bias)
